# baseline (speedup 1.0000x reference)
import functools

import jax
import jax.numpy as jnp
from jax import lax
from jax.experimental import pallas as pl
from jax.experimental.pallas import tpu as pltpu
from jax.experimental.pallas import tpu_sc as plsc

P = 77
D = 768
LANES = 16
NC = 2
NS = 16
NW = NC * NS
C = 32
NBUF = 4
DGROUP = 12


def _body(nch, logb, idx_hbm, tok_hbm, pos_hbm, out_hbm, idx_v, rows_v, pidx_v,
          pos16_v, gsem, ssem):
    wid = lax.axis_index("s") * NC + lax.axis_index("c")
    base = wid * (nch * C)

    pltpu.sync_copy(idx_hbm.at[pl.ds(base, nch * C)], idx_v)
    p0 = lax.shift_right_logical(base, logb)
    pidx_v[...] = lax.min(lax.iota(jnp.int32, LANES) + p0, P - 1)
    pltpu.async_copy(pos_hbm.at[pidx_v], pos16_v, gsem.at[0]).wait()

    def gather(c, b):
        return pltpu.make_async_copy(tok_hbm.at[idx_v.at[pl.ds(c * C, C)]],
                                     rows_v.at[b], gsem.at[b])

    def scatter(c, b):
        off = base + c * C
        return pltpu.make_async_copy(rows_v.at[b], out_hbm.at[pl.ds(off, C)],
                                     ssem.at[b])

    def add_pos(c, b):
        prow = lax.shift_right_logical(base + c * C, logb) - p0
        for g in range(D // (LANES * DGROUP)):
            sls = [pl.ds((g * DGROUP + q) * LANES, LANES) for q in range(DGROUP)]
            pv = [pos16_v[prow, sl] for sl in sls]

            def jbody(j, _):
                xs = [rows_v[b, j, sl] for sl in sls]
                for sl, x, pvd in zip(sls, xs, pv):
                    rows_v[b, j, sl] = x + pvd
                return 0
            lax.fori_loop(0, C, jbody, 0)

    gather(0, 0).start()

    def cc_body(cc, _):
        for b in range(NBUF):
            c = cc * NBUF + b
            nb = (b + 1) % NBUF

            gather(c, b).wait()
            add_pos(c, b)
            scatter(c, b).start()

            @pl.when(c + 1 < nch)
            def _():
                @pl.when(c + 1 >= NBUF)
                def _():
                    scatter(c + 1 - NBUF, nb).wait()
                gather(c + 1, nb).start()
        return 0

    lax.fori_loop(0, nch // NBUF, cc_body, 0)
    for k in range(NBUF):
        scatter(nch - NBUF + k, (nch - NBUF + k) % NBUF).wait()


def kernel(input_tokens, token_embedding, position_embedding):
    B, p = input_tokens.shape
    assert p == P and token_embedding.shape[1] == D
    assert B & (B - 1) == 0
    logb = B.bit_length() - 1
    N = B * P
    assert N % (NW * C) == 0 and B % C == 0
    nch = N // (NW * C)

    idx = input_tokens.T.reshape(N).astype(jnp.int32)
    mesh = plsc.VectorSubcoreMesh(core_axis_name="c", subcore_axis_name="s",
                                  num_cores=NC, num_subcores=NS)
    run = pl.kernel(
        functools.partial(_body, nch, logb),
        out_type=jax.ShapeDtypeStruct((N, D), jnp.float32),
        mesh=mesh,
        scratch_types=[
            pltpu.VMEM((nch * C,), jnp.int32),
            pltpu.VMEM((NBUF, C, D), jnp.float32),
            pltpu.VMEM((LANES,), jnp.int32),
            pltpu.VMEM((LANES, D), jnp.float32),
            pltpu.SemaphoreType.DMA((NBUF,)),
            pltpu.SemaphoreType.DMA((NBUF,)),
        ],
    )
    out = run(idx, token_embedding, position_embedding)
    return out.reshape(P, B, D).transpose(1, 0, 2)


# speedup vs baseline: 1.5468x; 1.5468x over previous
import functools

import jax
import jax.numpy as jnp
from jax import lax
from jax.experimental import pallas as pl
from jax.experimental.pallas import tpu as pltpu
from jax.experimental.pallas import tpu_sc as plsc

P = 77
D = 768
LANES = 16
NC = 2
NS = 16
NW = NC * NS
C = 32
NBUF = 4
DGROUP = 12


def _body(nch, logb, idx_hbm, tok_hbm, pos_hbm, out_hbm, idx_v, rows_v, pidx_v,
          pos16_v, gsem, ssem):
    wid = lax.axis_index("s") * NC + lax.axis_index("c")
    base = wid * (nch * C)

    pltpu.sync_copy(idx_hbm.at[pl.ds(base, nch * C)], idx_v)
    p0 = lax.shift_right_logical(base, logb)
    pidx_v[...] = lax.min(lax.iota(jnp.int32, LANES) + p0, P - 1)
    pltpu.async_copy(pos_hbm.at[pidx_v], pos16_v, gsem.at[0]).wait()

    def gather(c, b):
        return pltpu.make_async_copy(tok_hbm.at[idx_v.at[pl.ds(c * C, C)]],
                                     rows_v.at[b], gsem.at[b])

    def scatter(c, b):
        off = base + c * C
        return pltpu.make_async_copy(rows_v.at[b], out_hbm.at[pl.ds(off, C)],
                                     ssem.at[b])

    def add_pos(c, b):
        prow = lax.shift_right_logical(base + c * C, logb) - p0
        for g in range(D // (LANES * DGROUP)):
            sls = [pl.ds((g * DGROUP + q) * LANES, LANES) for q in range(DGROUP)]
            pv = [pos16_v[prow, sl] for sl in sls]

            def jbody(j, _):
                xs = [rows_v[b, j, sl] for sl in sls]
                for sl, x, pvd in zip(sls, xs, pv):
                    rows_v[b, j, sl] = x + pvd
                return 0
            lax.fori_loop(0, C, jbody, 0)

    gather(0, 0).start()

    def cc_body(cc, _):
        for b in range(NBUF):
            c = cc * NBUF + b
            nb = (b + 1) % NBUF

            @pl.when(c + 1 < nch)
            def _():
                @pl.when(c + 1 >= NBUF)
                def _():
                    scatter(c + 1 - NBUF, nb).wait()
                gather(c + 1, nb).start()

            gather(c, b).wait()
            add_pos(c, b)
            scatter(c, b).start()
        return 0

    lax.fori_loop(0, nch // NBUF, cc_body, 0)
    for k in range(NBUF):
        scatter(nch - NBUF + k, (nch - NBUF + k) % NBUF).wait()


def kernel(input_tokens, token_embedding, position_embedding):
    B, p = input_tokens.shape
    assert p == P and token_embedding.shape[1] == D
    assert B & (B - 1) == 0
    logb = B.bit_length() - 1
    N = B * P
    assert N % (NW * C) == 0 and B % C == 0
    nch = N // (NW * C)

    idx = input_tokens.T.reshape(N).astype(jnp.int32)
    mesh = plsc.VectorSubcoreMesh(core_axis_name="c", subcore_axis_name="s",
                                  num_cores=NC, num_subcores=NS)
    run = pl.kernel(
        functools.partial(_body, nch, logb),
        out_type=jax.ShapeDtypeStruct((N, D), jnp.float32),
        mesh=mesh,
        scratch_types=[
            pltpu.VMEM((nch * C,), jnp.int32),
            pltpu.VMEM((NBUF, C, D), jnp.float32),
            pltpu.VMEM((LANES,), jnp.int32),
            pltpu.VMEM((LANES, D), jnp.float32),
            pltpu.SemaphoreType.DMA((NBUF,)),
            pltpu.SemaphoreType.DMA((NBUF,)),
        ],
    )
    out = run(idx, token_embedding, position_embedding)
    return out.reshape(P, B, D).transpose(1, 0, 2)


# speedup vs baseline: 1.5515x; 1.0030x over previous
import functools

import jax
import jax.numpy as jnp
from jax import lax
from jax.experimental import pallas as pl
from jax.experimental.pallas import tpu as pltpu
from jax.experimental.pallas import tpu_sc as plsc

P = 77
D = 768
LANES = 16
NC = 2
NS = 16
NW = NC * NS
C = 32
NBUF = 4
DGROUP = 16


def _body(nch, logb, idx_hbm, tok_hbm, pos_hbm, out_hbm, idx_v, rows_v, pidx_v,
          pos16_v, gsem, ssem):
    wid = lax.axis_index("s") * NC + lax.axis_index("c")
    base = wid * (nch * C)

    pltpu.sync_copy(idx_hbm.at[pl.ds(base, nch * C)], idx_v)
    p0 = lax.shift_right_logical(base, logb)
    pidx_v[...] = lax.min(lax.iota(jnp.int32, LANES) + p0, P - 1)
    pltpu.async_copy(pos_hbm.at[pidx_v], pos16_v, gsem.at[0]).wait()

    def gather(c, b):
        return pltpu.make_async_copy(tok_hbm.at[idx_v.at[pl.ds(c * C, C)]],
                                     rows_v.at[b], gsem.at[b])

    def scatter(c, b):
        off = base + c * C
        return pltpu.make_async_copy(rows_v.at[b], out_hbm.at[pl.ds(off, C)],
                                     ssem.at[b])

    def add_pos(c, b):
        prow = lax.shift_right_logical(base + c * C, logb) - p0
        for g in range(D // (LANES * DGROUP)):
            sls = [pl.ds((g * DGROUP + q) * LANES, LANES) for q in range(DGROUP)]
            pv = [pos16_v[prow, sl] for sl in sls]

            def jbody(j, _):
                xs = [rows_v[b, j, sl] for sl in sls]
                for sl, x, pvd in zip(sls, xs, pv):
                    rows_v[b, j, sl] = x + pvd
                return 0
            lax.fori_loop(0, C, jbody, 0)

    gather(0, 0).start()

    def cc_body(cc, _):
        for b in range(NBUF):
            c = cc * NBUF + b
            nb = (b + 1) % NBUF

            @pl.when(c + 1 < nch)
            def _():
                @pl.when(c + 1 >= NBUF)
                def _():
                    scatter(c + 1 - NBUF, nb).wait()
                gather(c + 1, nb).start()

            gather(c, b).wait()
            add_pos(c, b)
            scatter(c, b).start()
        return 0

    lax.fori_loop(0, nch // NBUF, cc_body, 0)
    for k in range(NBUF):
        scatter(nch - NBUF + k, (nch - NBUF + k) % NBUF).wait()


def kernel(input_tokens, token_embedding, position_embedding):
    B, p = input_tokens.shape
    assert p == P and token_embedding.shape[1] == D
    assert B & (B - 1) == 0
    logb = B.bit_length() - 1
    N = B * P
    assert N % (NW * C) == 0 and B % C == 0
    nch = N // (NW * C)

    idx = input_tokens.T.reshape(N).astype(jnp.int32)
    mesh = plsc.VectorSubcoreMesh(core_axis_name="c", subcore_axis_name="s",
                                  num_cores=NC, num_subcores=NS)
    run = pl.kernel(
        functools.partial(_body, nch, logb),
        out_type=jax.ShapeDtypeStruct((N, D), jnp.float32),
        mesh=mesh,
        scratch_types=[
            pltpu.VMEM((nch * C,), jnp.int32),
            pltpu.VMEM((NBUF, C, D), jnp.float32),
            pltpu.VMEM((LANES,), jnp.int32),
            pltpu.VMEM((LANES, D), jnp.float32),
            pltpu.SemaphoreType.DMA((NBUF,)),
            pltpu.SemaphoreType.DMA((NBUF,)),
        ],
    )
    out = run(idx, token_embedding, position_embedding)
    return out.reshape(P, B, D).transpose(1, 0, 2)
